# baseline (speedup 1.0000x reference)
import functools

import jax
import jax.numpy as jnp
from jax import lax
from jax.experimental import pallas as pl
from jax.experimental.pallas import tpu as pltpu
from jax.experimental.pallas import tpu_sc as plsc

N = 10000
F = 128
B = 256
NC = 2
NS = 16
NW = NC * NS
RPW = 320
CHUNK = 64
NCHUNK = RPW // CHUNK
TAIL = N - (NW - 1) * RPW
TCH = TAIL // CHUNK
TREM = TAIL - TCH * CHUNK
SE = 384
NV = RPW // 16

_sc_mesh = plsc.VectorSubcoreMesh(core_axis_name="c", subcore_axis_name="s")


@functools.partial(
    pl.kernel,
    out_type=(
        jax.ShapeDtypeStruct((NC, B + 8, F), jnp.float32),
        jax.ShapeDtypeStruct((NW, SE), jnp.int32),
        jax.ShapeDtypeStruct((NW, SE), jnp.int32),
    ),
    mesh=_sc_mesh,
    compiler_params=pltpu.CompilerParams(needs_layout_passes=False,
                                         use_tc_tiling_on_sc=True),
    scratch_types=[
        pltpu.VMEM((RPW, F), jnp.float32),
        pltpu.VMEM((NCHUNK, CHUNK), jnp.int32),
        pltpu.VMEM((RPW + 16,), jnp.int32),
        pltpu.VMEM((SE,), jnp.int32),
        pltpu.VMEM((SE,), jnp.int32),
        pltpu.VMEM((16, F), jnp.float32),
        pltpu.VMEM_SHARED((B + 8, F), jnp.float32),
        pltpu.SemaphoreType.DMA,
    ],
)
def _sc_segsum(x_hbm, b_hbm, out_sum, out_st, out_en,
               rows_v, idx_v, ids_v, st_v, en_v, zb_v, acc_sh, sem):
    c = lax.axis_index("c")
    s = lax.axis_index("s")
    w = c * NS + s
    last = NW - 1

    @pl.when(w < last)
    def _start_rows():
        pltpu.async_copy(x_hbm.at[pl.ds(w * RPW, RPW)], rows_v, sem)

    @pl.when(w == last)
    def _start_rows_tail():
        pltpu.async_copy(x_hbm.at[pl.ds(last * RPW, TAIL)],
                         rows_v.at[pl.ds(0, TAIL)], sem)

    zeros16f = jnp.zeros((16,), jnp.float32)

    def _zfill(i, _):
        zb_v[i // (F // 16), pl.ds((i % (F // 16)) * 16, 16)] = zeros16f
        return 0

    lax.fori_loop(0, 16 * (F // 16), _zfill, 0)
    pltpu.sync_copy(zb_v, acc_sh.at[pl.ds(s * 16, 16)])

    @pl.when(w < last)
    def _load_ids():
        for j in range(NCHUNK):
            pltpu.sync_copy(b_hbm.at[pl.ds(w * RPW + j * CHUNK, CHUNK)],
                            idx_v.at[j])
        pltpu.sync_copy(b_hbm.at[pl.ds(w * RPW, RPW + 16)], ids_v)

    @pl.when(w == last)
    def _load_ids_tail():
        sent = jnp.full((16,), B, jnp.int32)

        def _fill_idx(k, _):
            idx_v[k // 4, pl.ds((k % 4) * 16, 16)] = sent
            return 0

        def _fill_ids(k, _):
            ids_v[pl.ds(k * 16, 16)] = sent
            return 0

        lax.fori_loop(0, NCHUNK * CHUNK // 16, _fill_idx, 0)
        lax.fori_loop(0, (RPW + 16) // 16, _fill_ids, 0)
        for j in range(TCH):
            pltpu.sync_copy(b_hbm.at[pl.ds(last * RPW + j * CHUNK, CHUNK)],
                            idx_v.at[j])
        pltpu.sync_copy(b_hbm.at[pl.ds(last * RPW + TCH * CHUNK, TREM)],
                        idx_v.at[TCH, pl.ds(0, TREM)])
        pltpu.sync_copy(b_hbm.at[pl.ds(last * RPW, TAIL)],
                        ids_v.at[pl.ds(0, TAIL)])

    zeros16 = jnp.zeros((16,), jnp.int32)

    def _zse(k, _):
        st_v[pl.ds(k * 16, 16)] = zeros16
        en_v[pl.ds(k * 16, 16)] = zeros16
        return 0

    lax.fori_loop(0, SE // 16, _zse, 0)
    iota = lax.iota(jnp.int32, 16)
    base = w * RPW

    def _bnd(p, _):
        idvec = ids_v[pl.ds(16 * p, 16)]
        idnext = ids_v[pl.ds(16 * p + 1, 16)]
        pos = iota + (base + 16 * p)
        m = idvec != idnext
        en_idx = jnp.where(m, idvec, SE - 1)
        st_idx = jnp.where(m, idnext, SE - 1)
        plsc.store_scatter(en_v, [en_idx], pos)
        plsc.store_scatter(st_v, [st_idx], pos + 1)
        return 0

    lax.fori_loop(0, NV, _bnd, 0)
    pltpu.sync_copy(st_v, out_st.at[w])
    pltpu.sync_copy(en_v, out_en.at[w])

    @pl.when(w < last)
    def _wait_rows():
        pltpu.make_async_copy(x_hbm.at[pl.ds(w * RPW, RPW)], rows_v, sem).wait()

    @pl.when(w == last)
    def _wait_rows_tail():
        pltpu.make_async_copy(x_hbm.at[pl.ds(last * RPW, TAIL)],
                              rows_v.at[pl.ds(0, TAIL)], sem).wait()

    plsc.subcore_barrier()
    for j in range(NCHUNK):
        pltpu.async_copy(rows_v.at[pl.ds(j * CHUNK, CHUNK)],
                         acc_sh.at[idx_v.at[j]], sem, add=True)
    for j in range(NCHUNK):
        pltpu.make_async_copy(rows_v.at[pl.ds(j * CHUNK, CHUNK)],
                              acc_sh.at[idx_v.at[j]], sem).wait()
    plsc.subcore_barrier()

    @pl.when(s == 0)
    def _push():
        pltpu.sync_copy(acc_sh, out_sum.at[c])


def _tc_mlp_body(ps_ref, st_ref, en_ref, u_ref, w1_ref, b1_ref,
                 w2_ref, b2_ref, w3_ref, b3_ref, o_ref):
    sums = ps_ref[0, 0:B, :] + ps_ref[1, 0:B, :]
    starts = jnp.sum(st_ref[:], axis=0)
    ends = jnp.sum(en_ref[:], axis=0)
    cnt = (ends - starts + 1).astype(jnp.float32)[0:B]
    recip = 1.0 / jnp.maximum(cnt, 1.0)
    eye = (lax.broadcasted_iota(jnp.int32, (B, B), 0) ==
           lax.broadcasted_iota(jnp.int32, (B, B), 1)).astype(jnp.float32)
    d = eye * recip
    xm = jnp.dot(d, sums, precision=lax.Precision.HIGHEST,
                 preferred_element_type=jnp.float32)
    h = jnp.dot(u_ref[:], w1_ref[0:6, :], preferred_element_type=jnp.float32)
    h = h + jnp.dot(xm, w1_ref[6:134, :], preferred_element_type=jnp.float32)
    h = jnp.maximum(h + b1_ref[:], 0.0)
    h = jnp.maximum(
        jnp.dot(h, w2_ref[:], preferred_element_type=jnp.float32) + b2_ref[:],
        0.0)
    o_ref[:] = (jnp.dot(h, w3_ref[:], preferred_element_type=jnp.float32)
                + b3_ref[:])


_tc_mlp = pl.pallas_call(
    _tc_mlp_body,
    out_shape=jax.ShapeDtypeStruct((B, 128), jnp.float32),
)


def kernel(x, edge_index, edge_attr, u, batch, W1, b1, W2, b2, W3, b3):
    del edge_index, edge_attr
    psum, st, en = _sc_segsum(x, batch)
    return _tc_mlp(psum, st, en, u, W1, b1.reshape(1, -1), W2,
                   b2.reshape(1, -1), W3, b3.reshape(1, -1))


# speedup vs baseline: 1.0080x; 1.0080x over previous
import functools

import jax
import jax.numpy as jnp
from jax import lax
from jax.experimental import pallas as pl
from jax.experimental.pallas import tpu as pltpu
from jax.experimental.pallas import tpu_sc as plsc

N = 10000
F = 128
B = 256
NC = 2
NS = 16
NW = NC * NS
RPW = 320
CHUNK = 64
NCHUNK = RPW // CHUNK
TAIL = N - (NW - 1) * RPW
TCH = TAIL // CHUNK
TREM = TAIL - TCH * CHUNK
SE = 384
NV = RPW // 16

_sc_mesh = plsc.VectorSubcoreMesh(core_axis_name="c", subcore_axis_name="s")


@functools.partial(
    pl.kernel,
    out_type=(
        jax.ShapeDtypeStruct((NC, B + 8, F), jnp.float32),
        jax.ShapeDtypeStruct((NW, SE), jnp.int32),
        jax.ShapeDtypeStruct((NW, SE), jnp.int32),
    ),
    mesh=_sc_mesh,
    compiler_params=pltpu.CompilerParams(needs_layout_passes=False),
    scratch_types=[
        pltpu.VMEM((RPW, F), jnp.float32),
        pltpu.VMEM((NCHUNK, CHUNK), jnp.int32),
        pltpu.VMEM((RPW + 16,), jnp.int32),
        pltpu.VMEM((SE,), jnp.int32),
        pltpu.VMEM((SE,), jnp.int32),
        pltpu.VMEM((16, F), jnp.float32),
        pltpu.VMEM_SHARED((B + 8, F), jnp.float32),
        pltpu.SemaphoreType.DMA,
    ],
)
def _sc_segsum(x_hbm, b_hbm, out_sum, out_st, out_en,
               rows_v, idx_v, ids_v, st_v, en_v, zb_v, acc_sh, sem):
    c = lax.axis_index("c")
    s = lax.axis_index("s")
    w = c * NS + s
    last = NW - 1

    @pl.when(w < last)
    def _start_rows():
        pltpu.async_copy(x_hbm.at[pl.ds(w * RPW, RPW)], rows_v, sem)

    @pl.when(w == last)
    def _start_rows_tail():
        pltpu.async_copy(x_hbm.at[pl.ds(last * RPW, TAIL)],
                         rows_v.at[pl.ds(0, TAIL)], sem)

    zeros16f = jnp.zeros((16,), jnp.float32)

    def _zfill(i, _):
        zb_v[i // (F // 16), pl.ds((i % (F // 16)) * 16, 16)] = zeros16f
        return 0

    lax.fori_loop(0, 16 * (F // 16), _zfill, 0)
    pltpu.sync_copy(zb_v, acc_sh.at[pl.ds(s * 16, 16)])

    @pl.when(w < last)
    def _load_ids():
        for j in range(NCHUNK):
            pltpu.sync_copy(b_hbm.at[pl.ds(w * RPW + j * CHUNK, CHUNK)],
                            idx_v.at[j])
        pltpu.sync_copy(b_hbm.at[pl.ds(w * RPW, RPW + 16)], ids_v)

    @pl.when(w == last)
    def _load_ids_tail():
        sent = jnp.full((16,), B, jnp.int32)

        def _fill_idx(k, _):
            idx_v[k // 4, pl.ds((k % 4) * 16, 16)] = sent
            return 0

        def _fill_ids(k, _):
            ids_v[pl.ds(k * 16, 16)] = sent
            return 0

        lax.fori_loop(0, NCHUNK * CHUNK // 16, _fill_idx, 0)
        lax.fori_loop(0, (RPW + 16) // 16, _fill_ids, 0)
        for j in range(TCH):
            pltpu.sync_copy(b_hbm.at[pl.ds(last * RPW + j * CHUNK, CHUNK)],
                            idx_v.at[j])
        pltpu.sync_copy(b_hbm.at[pl.ds(last * RPW + TCH * CHUNK, TREM)],
                        idx_v.at[TCH, pl.ds(0, TREM)])
        pltpu.sync_copy(b_hbm.at[pl.ds(last * RPW, TAIL)],
                        ids_v.at[pl.ds(0, TAIL)])

    zeros16 = jnp.zeros((16,), jnp.int32)

    def _zse(k, _):
        st_v[pl.ds(k * 16, 16)] = zeros16
        en_v[pl.ds(k * 16, 16)] = zeros16
        return 0

    lax.fori_loop(0, SE // 16, _zse, 0)
    iota = lax.iota(jnp.int32, 16)
    base = w * RPW

    def _bnd(p, _):
        idvec = ids_v[pl.ds(16 * p, 16)]
        idnext = ids_v[pl.ds(16 * p + 1, 16)]
        pos = iota + (base + 16 * p)
        m = idvec != idnext
        en_idx = jnp.where(m, idvec, SE - 1)
        st_idx = jnp.where(m, idnext, SE - 1)
        plsc.store_scatter(en_v, [en_idx], pos)
        plsc.store_scatter(st_v, [st_idx], pos + 1)
        return 0

    lax.fori_loop(0, NV, _bnd, 0)
    pltpu.sync_copy(st_v, out_st.at[w])
    pltpu.sync_copy(en_v, out_en.at[w])

    @pl.when(w < last)
    def _wait_rows():
        pltpu.make_async_copy(x_hbm.at[pl.ds(w * RPW, RPW)], rows_v, sem).wait()

    @pl.when(w == last)
    def _wait_rows_tail():
        pltpu.make_async_copy(x_hbm.at[pl.ds(last * RPW, TAIL)],
                              rows_v.at[pl.ds(0, TAIL)], sem).wait()

    plsc.subcore_barrier()
    for j in range(NCHUNK):
        pltpu.async_copy(rows_v.at[pl.ds(j * CHUNK, CHUNK)],
                         acc_sh.at[idx_v.at[j]], sem, add=True)
    for j in range(NCHUNK):
        pltpu.make_async_copy(rows_v.at[pl.ds(j * CHUNK, CHUNK)],
                              acc_sh.at[idx_v.at[j]], sem).wait()
    plsc.subcore_barrier()

    @pl.when(s == 0)
    def _push():
        pltpu.sync_copy(acc_sh, out_sum.at[c])


def _tc_mlp_body(ps_ref, st_ref, en_ref, u_ref, w1_ref, b1_ref,
                 w2_ref, b2_ref, w3_ref, b3_ref, o_ref):
    sums = ps_ref[0, 0:B, :] + ps_ref[1, 0:B, :]
    starts = jnp.sum(st_ref[:], axis=0)
    ends = jnp.sum(en_ref[:], axis=0)
    cnt = (ends - starts + 1).astype(jnp.float32)[0:B]
    recip = 1.0 / jnp.maximum(cnt, 1.0)
    eye = (lax.broadcasted_iota(jnp.int32, (B, B), 0) ==
           lax.broadcasted_iota(jnp.int32, (B, B), 1)).astype(jnp.float32)
    d = eye * recip
    xm = jnp.dot(d, sums, precision=lax.Precision.HIGHEST,
                 preferred_element_type=jnp.float32)
    h = lax.dot_general(u_ref[:], w1_ref[0:6, :], (((0,), (0,)), ((), ())),
                        preferred_element_type=jnp.float32)
    h = h + jnp.dot(xm, w1_ref[6:134, :], preferred_element_type=jnp.float32)
    h = jnp.maximum(h + b1_ref[:], 0.0)
    h = jnp.maximum(
        jnp.dot(h, w2_ref[:], preferred_element_type=jnp.float32) + b2_ref[:],
        0.0)
    o_ref[:] = (jnp.dot(h, w3_ref[:], preferred_element_type=jnp.float32)
                + b3_ref[:])


_tc_mlp = pl.pallas_call(
    _tc_mlp_body,
    out_shape=jax.ShapeDtypeStruct((B, 128), jnp.float32),
)


def kernel(x, edge_index, edge_attr, u, batch, W1, b1, W2, b2, W3, b3):
    del edge_index, edge_attr
    psum, st, en = _sc_segsum(x, batch)
    return _tc_mlp(psum, st, en, u.T, W1, b1.reshape(1, -1), W2,
                   b2.reshape(1, -1), W3, b3.reshape(1, -1))
